# baseline (speedup 1.0000x reference)
import functools

import jax
import jax.numpy as jnp
from jax import lax
from jax.experimental import pallas as pl
from jax.experimental.pallas import tpu as pltpu
from jax.experimental.pallas import tpu_sc as plsc

B = 16384
D = 64
K = 4
NC = 2
NS = 16
NW = NC * NS
B_PER_W = B // NW
CHUNK = 128
N_CHUNKS = B_PER_W // CHUNK

_DNUMS = lax.GatherDimensionNumbers(
    offset_dims=(), collapsed_slice_dims=(0,), start_index_map=(0,))


def _bcast_lane(v, lane):
    idx = jnp.full((16,), lane, jnp.int32)
    return lax.gather(v, idx[:, None], _DNUMS, (1,),
                      mode=lax.GatherScatterMode.PROMISE_IN_BOUNDS)


def _eges_body(idx_hbm, es_hbm, eb_hbm, eh_hbm, ec_hbm, at_hbm, out_hbm,
               idx_v, aidx_v, rows_v, logit_v, out_v, sem):
    wid = lax.axis_index("s") * NC + lax.axis_index("c")
    base = wid * B_PER_W

    for f in range(K):
        pltpu.sync_copy(idx_hbm.at[f, pl.ds(base, B_PER_W)], idx_v.at[f])

    def aidx(i, carry):
        sv = idx_v[0, pl.ds(i * 16, 16)] * K
        for k in range(K):
            aidx_v[k, pl.ds(i * 16, 16)] = sv + k
        return carry

    lax.fori_loop(0, B_PER_W // 16, aidx, 0)

    tables = (es_hbm, eb_hbm, eh_hbm, ec_hbm)

    for c in range(N_CHUNKS):
        off = c * CHUNK
        copies = []
        for f in range(K):
            copies.append(pltpu.async_copy(
                tables[f].at[idx_v.at[f, pl.ds(off, CHUNK)]],
                rows_v.at[f], sem))
        for k in range(K):
            copies.append(pltpu.async_copy(
                at_hbm.at[aidx_v.at[k, pl.ds(off, CHUNK)]],
                logit_v.at[k], sem))
        for cp in copies:
            cp.wait()

        def group(g, carry):
            tbase = g * 16
            e = [jnp.exp(logit_v[k, pl.ds(tbase, 16)]) for k in range(K)]
            s = (e[0] + e[1]) + (e[2] + e[3])
            w16 = [e[k] / s for k in range(K)]
            for l in range(16):
                t = tbase + l
                acc = [None] * (D // 16)
                for k in range(K):
                    wk = _bcast_lane(w16[k], l)
                    for j in range(D // 16):
                        term = rows_v[k, t, pl.ds(j * 16, 16)] * wk
                        acc[j] = term if k == 0 else acc[j] + term
                for j in range(D // 16):
                    out_v[t, pl.ds(j * 16, 16)] = acc[j]
            return carry

        lax.fori_loop(0, CHUNK // 16, group, 0)

        pltpu.sync_copy(out_v, out_hbm.at[pl.ds(base + off, CHUNK)])


@jax.jit
def _eges(idx, emb_sku, emb_brand, emb_shop, emb_cate, attn_flat):
    return pl.kernel(
        _eges_body,
        mesh=plsc.VectorSubcoreMesh(core_axis_name="c", subcore_axis_name="s"),
        compiler_params=pltpu.CompilerParams(use_tc_tiling_on_sc=False),
        out_type=jax.ShapeDtypeStruct((B, D), jnp.float32),
        scratch_types=[
            pltpu.VMEM((K, B_PER_W), jnp.int32),
            pltpu.VMEM((K, B_PER_W), jnp.int32),
            pltpu.VMEM((K, CHUNK, D), jnp.float32),
            pltpu.VMEM((K, CHUNK), jnp.float32),
            pltpu.VMEM((CHUNK, D), jnp.float32),
            pltpu.SemaphoreType.DMA,
        ],
    )(idx, emb_sku, emb_brand, emb_shop, emb_cate, attn_flat)


def kernel(sku_id, brand, shop, cate, emb_sku, emb_brand, emb_shop, emb_cate,
           attn_tab):
    idx = jnp.stack([sku_id, brand, shop, cate]).astype(jnp.int32)
    attn_flat = attn_tab.reshape(-1)
    return _eges(idx, emb_sku, emb_brand, emb_shop, emb_cate, attn_flat)
